# Initial kernel scaffold; baseline (speedup 1.0000x reference)
import jax
import jax.numpy as jnp
from jax.experimental import pallas as pl


def kernel(img):
    raise NotImplementedError("write your pallas kernel here")


# speedup vs baseline: 210.6821x; 210.6821x over previous
import math

import jax
import jax.numpy as jnp
from jax.experimental import pallas as pl

NB = 10
PPC = 8


def _hog_body(x_ref, o_ref):
    H, W = x_ref.shape[1:]
    CH, CW = H // PPC, W // PPC
    img = x_ref[0].astype(jnp.bfloat16).astype(jnp.float32)

    z_row = jnp.zeros((1, W), jnp.float32)
    z_col = jnp.zeros((H, 1), jnp.float32)

    up = jnp.concatenate([z_row, img[:-1]], axis=0)
    dn = jnp.concatenate([img[1:], z_row], axis=0)
    c = up + dn + img + img
    gx = (jnp.concatenate([z_col, c[:, :-1]], axis=1)
          - jnp.concatenate([c[:, 1:], z_col], axis=1))

    lf = jnp.concatenate([z_col, img[:, :-1]], axis=1)
    rt = jnp.concatenate([img[:, 1:], z_col], axis=1)
    d = lf + rt + img + img
    gy = (jnp.concatenate([z_row, d[:-1]], axis=0)
          - jnp.concatenate([d[1:], z_row], axis=0))

    norm = jnp.sqrt(gx * gx + gy * gy)

    phase = jnp.arctan2(gx, gy)
    t = phase / math.pi * NB
    fl = jnp.mod(jnp.floor(t), NB).astype(jnp.int32)
    ce = jnp.mod(jnp.ceil(t), NB).astype(jnp.int32)

    ri = jax.lax.broadcasted_iota(jnp.int32, (W, CW), 0)
    cj = jax.lax.broadcasted_iota(jnp.int32, (W, CW), 1)
    pool_m = jnp.where(ri // PPC == cj, 1.0 / (PPC * PPC), 0.0)

    one_m = 1.0 - norm
    for k in range(NB):
        val = (jnp.where(fl == k, norm, 0.0)
               + jnp.where(ce == k, one_m, 0.0))
        vp = val.reshape(CH, PPC, W).sum(axis=1)
        o_ref[0, k] = jax.lax.dot_general(
            vp, pool_m, (((1,), (0,)), ((), ())),
            precision=jax.lax.Precision.HIGHEST,
            preferred_element_type=jnp.float32)


def kernel(img):
    n, _, H, W = img.shape
    x = img.reshape(n, H, W)
    CH, CW = H // PPC, W // PPC
    out = pl.pallas_call(
        _hog_body,
        grid=(n,),
        in_specs=[pl.BlockSpec((1, H, W), lambda i: (i, 0, 0))],
        out_specs=pl.BlockSpec((1, NB, CH, CW), lambda i: (i, 0, 0, 0)),
        out_shape=jax.ShapeDtypeStruct((n, NB, CH, CW), jnp.float32),
    )(x)
    return out.reshape(n, -1)


# speedup vs baseline: 322.7572x; 1.5320x over previous
import math

import jax
import jax.numpy as jnp
from jax.experimental import pallas as pl

NB = 10
PPC = 8


def _hog_body(x_ref, o_ref):
    H, W = x_ref.shape[1:]
    CH, CW = H // PPC, W // PPC
    img = x_ref[0].astype(jnp.bfloat16).astype(jnp.float32)

    z_row = jnp.zeros((1, W), jnp.float32)
    z_col = jnp.zeros((H, 1), jnp.float32)

    up = jnp.concatenate([z_row, img[:-1]], axis=0)
    dn = jnp.concatenate([img[1:], z_row], axis=0)
    c = up + dn + img + img
    gx = (jnp.concatenate([z_col, c[:, :-1]], axis=1)
          - jnp.concatenate([c[:, 1:], z_col], axis=1))

    lf = jnp.concatenate([z_col, img[:, :-1]], axis=1)
    rt = jnp.concatenate([img[:, 1:], z_col], axis=1)
    d = lf + rt + img + img
    gy = (jnp.concatenate([z_row, d[:-1]], axis=0)
          - jnp.concatenate([d[1:], z_row], axis=0))

    norm = jnp.sqrt(gx * gx + gy * gy)

    phase = jnp.arctan2(gx, gy)
    t = phase / math.pi * NB
    flf = jnp.floor(t)
    flf = jnp.where(flf < 0.0, flf + NB, flf)
    flf = jnp.where(flf >= NB, flf - NB, flf)
    cef = jnp.ceil(t)
    cef = jnp.where(cef < 0.0, cef + NB, cef)
    cef = jnp.where(cef >= NB, cef - NB, cef)

    ri = jax.lax.broadcasted_iota(jnp.int32, (H, CH), 0)
    cj = jax.lax.broadcasted_iota(jnp.int32, (H, CH), 1)
    pool_v = jnp.where(ri // PPC == cj, 1.0 / PPC, 0.0)
    pool_h = pool_v

    one_m = 1.0 - norm
    for k in range(NB):
        val = (jnp.where(flf == k, norm, 0.0)
               + jnp.where(cef == k, one_m, 0.0))
        vp = jax.lax.dot_general(
            pool_v, val, (((0,), (0,)), ((), ())),
            preferred_element_type=jnp.float32)
        o_ref[0, k] = jax.lax.dot_general(
            vp, pool_h, (((1,), (0,)), ((), ())),
            preferred_element_type=jnp.float32)


def kernel(img):
    n, _, H, W = img.shape
    x = img.reshape(n, H, W)
    CH, CW = H // PPC, W // PPC
    out = pl.pallas_call(
        _hog_body,
        grid=(n,),
        in_specs=[pl.BlockSpec((1, H, W), lambda i: (i, 0, 0))],
        out_specs=pl.BlockSpec((1, NB, CH, CW), lambda i: (i, 0, 0, 0)),
        out_shape=jax.ShapeDtypeStruct((n, NB, CH, CW), jnp.float32),
    )(x)
    return out.reshape(n, -1)


# speedup vs baseline: 343.3953x; 1.0639x over previous
import math

import jax
import jax.numpy as jnp
from jax.experimental import pallas as pl

NB = 10
PPC = 8


def _hog_body(x_ref, o_ref):
    H, W = x_ref.shape[1:]
    CH, CW = H // PPC, W // PPC
    img = x_ref[0].astype(jnp.bfloat16).astype(jnp.float32)

    z_row = jnp.zeros((1, W), jnp.float32)
    z_col = jnp.zeros((H, 1), jnp.float32)

    up = jnp.concatenate([z_row, img[:-1]], axis=0)
    dn = jnp.concatenate([img[1:], z_row], axis=0)
    c = up + dn + img + img
    gx = (jnp.concatenate([z_col, c[:, :-1]], axis=1)
          - jnp.concatenate([c[:, 1:], z_col], axis=1))

    lf = jnp.concatenate([z_col, img[:, :-1]], axis=1)
    rt = jnp.concatenate([img[:, 1:], z_col], axis=1)
    d = lf + rt + img + img
    gy = (jnp.concatenate([z_row, d[:-1]], axis=0)
          - jnp.concatenate([d[1:], z_row], axis=0))

    norm = jnp.sqrt(gx * gx + gy * gy)

    phase = jnp.arctan2(gx, gy)
    t = phase / math.pi * NB
    bf = jnp.bfloat16
    flf = jnp.floor(t).astype(bf)
    flf = jnp.where(flf < 0.0, flf + NB, flf)
    flf = jnp.where(flf >= NB, flf - NB, flf)
    cef = jnp.ceil(t).astype(bf)
    cef = jnp.where(cef < 0.0, cef + NB, cef)
    cef = jnp.where(cef >= NB, cef - NB, cef)

    ri = jax.lax.broadcasted_iota(jnp.int32, (H, CH), 0)
    cj = jax.lax.broadcasted_iota(jnp.int32, (H, CH), 1)
    pool_v = jnp.where(ri // PPC == cj, 1.0 / PPC, 0.0).astype(bf)
    pool_h = pool_v

    norm_b = norm.astype(bf)
    one_m = (1.0 - norm).astype(bf)
    zb = jnp.zeros(norm.shape, bf)
    for k in range(NB):
        val = (jnp.where(flf == k, norm_b, zb)
               + jnp.where(cef == k, one_m, zb))
        vp = jax.lax.dot_general(
            pool_v, val, (((0,), (0,)), ((), ())),
            preferred_element_type=jnp.float32)
        o_ref[0, k] = jax.lax.dot_general(
            vp.astype(bf), pool_h, (((1,), (0,)), ((), ())),
            preferred_element_type=jnp.float32)


def kernel(img):
    n, _, H, W = img.shape
    x = img.reshape(n, H, W)
    CH, CW = H // PPC, W // PPC
    out = pl.pallas_call(
        _hog_body,
        grid=(n,),
        in_specs=[pl.BlockSpec((1, H, W), lambda i: (i, 0, 0))],
        out_specs=pl.BlockSpec((1, NB, CH, CW), lambda i: (i, 0, 0, 0)),
        out_shape=jax.ShapeDtypeStruct((n, NB, CH, CW), jnp.float32),
    )(x)
    return out.reshape(n, -1)


# speedup vs baseline: 349.1733x; 1.0168x over previous
import math

import jax
import jax.numpy as jnp
from jax.experimental import pallas as pl
from jax.experimental.pallas import tpu as pltpu

NB = 10
PPC = 8


def _hog_body(x_ref, band_ref, pool_ref, o_ref):
    H, W = x_ref.shape[1:]
    CH, CW = H // PPC, W // PPC
    bf = jnp.bfloat16

    img_b = x_ref[0].astype(bf)
    band = band_ref[...]
    pool = pool_ref[...]

    c = jax.lax.dot_general(
        band, img_b, (((1,), (0,)), ((), ())),
        preferred_element_type=jnp.float32)
    d = jax.lax.dot_general(
        img_b, band, (((1,), (0,)), ((), ())),
        preferred_element_type=jnp.float32)

    z_row = jnp.zeros((1, W), jnp.float32)
    z_col = jnp.zeros((H, 1), jnp.float32)
    gx = (jnp.concatenate([z_col, c[:, :-1]], axis=1)
          - jnp.concatenate([c[:, 1:], z_col], axis=1))
    gy = (jnp.concatenate([z_row, d[:-1]], axis=0)
          - jnp.concatenate([d[1:], z_row], axis=0))

    norm = jnp.sqrt(gx * gx + gy * gy)

    phase = jnp.arctan2(gx, gy)
    t = phase / math.pi * NB
    flf = jnp.floor(t).astype(bf)
    flf = jnp.where(flf < 0.0, flf + NB, flf)
    flf = jnp.where(flf >= NB, flf - NB, flf)
    cef = jnp.ceil(t).astype(bf)
    cef = jnp.where(cef < 0.0, cef + NB, cef)
    cef = jnp.where(cef >= NB, cef - NB, cef)

    norm_b = norm.astype(bf)
    one_m = (1.0 - norm).astype(bf)
    zb = jnp.zeros(norm.shape, bf)
    for k in range(NB):
        val = (jnp.where(flf == k, norm_b, zb)
               + jnp.where(cef == k, one_m, zb))
        vp = jax.lax.dot_general(
            pool, val, (((0,), (0,)), ((), ())),
            preferred_element_type=jnp.float32)
        o_ref[0, k] = jax.lax.dot_general(
            vp.astype(bf), pool, (((1,), (0,)), ((), ())),
            preferred_element_type=jnp.float32)


def kernel(img):
    n, _, H, W = img.shape
    x = img.reshape(n, H, W)
    CH, CW = H // PPC, W // PPC

    i = jnp.arange(H)
    diff = i[:, None] - i[None, :]
    band = (jnp.where(jnp.abs(diff) == 1, 1.0, 0.0)
            + jnp.where(diff == 0, 2.0, 0.0)).astype(jnp.bfloat16)
    pool = jnp.where(i[:, None] // PPC == jnp.arange(CH)[None, :],
                     1.0 / PPC, 0.0).astype(jnp.bfloat16)

    out = pl.pallas_call(
        _hog_body,
        grid=(n,),
        in_specs=[
            pl.BlockSpec((1, H, W), lambda i: (i, 0, 0)),
            pl.BlockSpec((H, H), lambda i: (0, 0)),
            pl.BlockSpec((H, CH), lambda i: (0, 0)),
        ],
        out_specs=pl.BlockSpec((1, NB, CH, CW), lambda i: (i, 0, 0, 0)),
        out_shape=jax.ShapeDtypeStruct((n, NB, CH, CW), jnp.float32),
    )(x, band, pool)
    return out.reshape(n, -1)


# speedup vs baseline: 361.3990x; 1.0350x over previous
import math

import jax
import jax.numpy as jnp
from jax.experimental import pallas as pl
from jax.experimental.pallas import tpu as pltpu

NB = 10
PPC = 8


def _hog_body(x_ref, band_ref, pool_ref, o_ref):
    H, W = x_ref.shape[1:]
    CH, CW = H // PPC, W // PPC
    bf = jnp.bfloat16

    img_b = x_ref[0].astype(bf)
    band = band_ref[...]
    pool = pool_ref[...]

    c = jax.lax.dot_general(
        band, img_b, (((1,), (0,)), ((), ())),
        preferred_element_type=jnp.float32)
    d = jax.lax.dot_general(
        img_b, band, (((1,), (0,)), ((), ())),
        preferred_element_type=jnp.float32)

    z_row = jnp.zeros((1, W), jnp.float32)
    z_col = jnp.zeros((H, 1), jnp.float32)
    gx = (jnp.concatenate([z_col, c[:, :-1]], axis=1)
          - jnp.concatenate([c[:, 1:], z_col], axis=1))
    gy = (jnp.concatenate([z_row, d[:-1]], axis=0)
          - jnp.concatenate([d[1:], z_row], axis=0))

    norm = jnp.sqrt(gx * gx + gy * gy)

    phase = jnp.arctan2(gx, gy)
    t = phase / math.pi * NB
    flf = jnp.floor(t).astype(bf)
    flf = jnp.where(flf < 0.0, flf + NB, flf)
    flf = jnp.where(flf >= NB, flf - NB, flf)
    cef = jnp.ceil(t).astype(bf)
    cef = jnp.where(cef < 0.0, cef + NB, cef)
    cef = jnp.where(cef >= NB, cef - NB, cef)

    norm_b = norm.astype(bf)
    one_m = (1.0 - norm).astype(bf)
    zb = jnp.zeros(norm.shape, bf)
    acc = None
    for k in range(NB - 1):
        val = (jnp.where(flf == k, norm_b, zb)
               + jnp.where(cef == k, one_m, zb))
        vp = jax.lax.dot_general(
            pool, val, (((0,), (0,)), ((), ())),
            preferred_element_type=jnp.float32)
        ok = jax.lax.dot_general(
            vp.astype(bf), pool, (((1,), (0,)), ((), ())),
            preferred_element_type=jnp.float32)
        o_ref[0, k] = ok
        acc = ok if acc is None else acc + ok
    o_ref[0, NB - 1] = 1.0 - acc


def kernel(img):
    n, _, H, W = img.shape
    x = img.reshape(n, H, W)
    CH, CW = H // PPC, W // PPC

    i = jnp.arange(H)
    diff = i[:, None] - i[None, :]
    band = (jnp.where(jnp.abs(diff) == 1, 1.0, 0.0)
            + jnp.where(diff == 0, 2.0, 0.0)).astype(jnp.bfloat16)
    pool = jnp.where(i[:, None] // PPC == jnp.arange(CH)[None, :],
                     1.0 / PPC, 0.0).astype(jnp.bfloat16)

    out = pl.pallas_call(
        _hog_body,
        grid=(n,),
        in_specs=[
            pl.BlockSpec((1, H, W), lambda i: (i, 0, 0)),
            pl.BlockSpec((H, H), lambda i: (0, 0)),
            pl.BlockSpec((H, CH), lambda i: (0, 0)),
        ],
        out_specs=pl.BlockSpec((1, NB, CH, CW), lambda i: (i, 0, 0, 0)),
        out_shape=jax.ShapeDtypeStruct((n, NB, CH, CW), jnp.float32),
    )(x, band, pool)
    return out.reshape(n, -1)
